# baseline (speedup 1.0000x reference)
import jax
import jax.numpy as jnp
from jax import lax
from jax.experimental import pallas as pl
from jax.experimental.pallas import tpu as pltpu
from jax.experimental.pallas import tpu_sc as plsc

N_FIELDS = 26
VOCAB = 100000
HIDDEN = 32
BATCH = 16384

NUM_CORES = 2
NUM_SUBCORES = 16
NW = NUM_CORES * NUM_SUBCORES
TOTAL = N_FIELDS * BATCH
RPW = TOTAL // NW
G = 1024
NCH = RPW // G
SLICE = BATCH // NUM_SUBCORES
LANES = 16


def _body(idx_hbm, pos_hbm, tab_hbm, part_hbm, idx_v, pos_v, rows_v, shared,
          sem0, sem1):
    cid = lax.axis_index("c")
    sid = lax.axis_index("s")
    wid = cid * NUM_SUBCORES + sid
    sems = [sem0, sem1]

    z = jnp.zeros((LANES,), jnp.float32)

    def zero_body(j, carry):
        rows_v[0, j, 0:16] = z
        rows_v[0, j, 16:32] = z
        return carry

    lax.fori_loop(0, SLICE, zero_body, 0)
    pltpu.sync_copy(rows_v.at[0], shared.at[pl.ds(sid * SLICE, SLICE)])
    plsc.subcore_barrier()

    def load_and_fire(t, b):
        pltpu.sync_copy(idx_hbm.at[wid, t], idx_v.at[b])
        pltpu.sync_copy(pos_hbm.at[wid, t], pos_v.at[b])
        return pltpu.async_copy(tab_hbm.at[idx_v.at[b]], rows_v.at[b], sems[b])

    handle = load_and_fire(0, 0)
    for t in range(NCH):
        b = t % 2
        nxt = None
        if t + 1 < NCH:
            nxt = load_and_fire(t + 1, 1 - b)
        handle.wait()
        pltpu.sync_copy(rows_v.at[b], shared.at[pos_v.at[b]], add=True)
        handle = nxt

    plsc.subcore_barrier()
    pltpu.sync_copy(shared.at[pl.ds(sid * SLICE, SLICE)],
                    part_hbm.at[cid, pl.ds(sid * SLICE, SLICE)])


_mesh = plsc.VectorSubcoreMesh(core_axis_name="c", subcore_axis_name="s")

_sc_call = pl.kernel(
    _body,
    out_type=jax.ShapeDtypeStruct((NUM_CORES, BATCH, HIDDEN), jnp.float32),
    mesh=_mesh,
    scratch_types=[
        pltpu.VMEM((2, G), jnp.int32),
        pltpu.VMEM((2, G), jnp.int32),
        pltpu.VMEM((2, G, HIDDEN), jnp.float32),
        pltpu.VMEM_SHARED((BATCH, HIDDEN), jnp.float32),
        pltpu.SemaphoreType.DMA,
        pltpu.SemaphoreType.DMA,
    ],
    compiler_params=pltpu.CompilerParams(use_tc_tiling_on_sc=False),
)


def _add_body(p_ref, o_ref):
    o_ref[...] = p_ref[0] + p_ref[1]


_tc_add = pl.pallas_call(
    _add_body,
    out_shape=jax.ShapeDtypeStruct((BATCH, HIDDEN), jnp.float32),
    in_specs=[pl.BlockSpec((NUM_CORES, 2048, HIDDEN), lambda i: (0, i, 0))],
    out_specs=pl.BlockSpec((2048, HIDDEN), lambda i: (i, 0)),
    grid=(BATCH // 2048,),
)


def kernel(x, tables):
    offs = (jnp.arange(N_FIELDS, dtype=jnp.int32) * VOCAB)[:, None]
    keys = x.astype(jnp.int32) + offs
    pos = jnp.broadcast_to(jnp.arange(BATCH, dtype=jnp.int32),
                           (N_FIELDS, BATCH))
    s_idx, s_pos = jax.lax.sort_key_val(keys, pos, dimension=1)
    s_idx = s_idx.reshape(NW, NCH, G)
    s_pos = s_pos.reshape(NW, NCH, G)
    tab_flat = tables.reshape(N_FIELDS * VOCAB, HIDDEN)
    partials = _sc_call(s_idx, s_pos, tab_flat)
    return _tc_add(partials)


# speedup vs baseline: 1.0180x; 1.0180x over previous
import jax
import jax.numpy as jnp
from jax import lax
from jax.experimental import pallas as pl
from jax.experimental.pallas import tpu as pltpu
from jax.experimental.pallas import tpu_sc as plsc

N_FIELDS = 26
VOCAB = 100000
HIDDEN = 32
BATCH = 16384

NUM_CORES = 2
NUM_SUBCORES = 16
NW = NUM_CORES * NUM_SUBCORES
BPW = BATCH // NW
CHUNK = 64
NCHUNK = BPW // CHUNK
ROWS = CHUNK * N_FIELDS
LANES = 16


def _body(x_hbm, tab_hbm, out_hbm, idx_v, rows_v, out_v, sem):
    wid = lax.axis_index("s") * NUM_CORES + lax.axis_index("c")

    def chunk_body(t, carry):
        pltpu.sync_copy(x_hbm.at[:, wid, t], idx_v)
        handles = []
        for i in range(N_FIELDS):
            handles.append(pltpu.async_copy(
                tab_hbm.at[i].at[idx_v.at[i]],
                rows_v.at[pl.ds(i * CHUNK, CHUNK)],
                sem))
        for h in handles:
            h.wait()

        def elem_body(c, carry2):
            a0 = rows_v[c, 0:16]
            a1 = rows_v[c, 16:32]
            for i in range(1, N_FIELDS):
                a0 = a0 + rows_v[i * CHUNK + c, 0:16]
                a1 = a1 + rows_v[i * CHUNK + c, 16:32]
            out_v[c, 0:16] = a0
            out_v[c, 16:32] = a1
            return carry2

        lax.fori_loop(0, CHUNK, elem_body, 0)
        pltpu.sync_copy(out_v, out_hbm.at[pl.ds(wid * BPW + t * CHUNK, CHUNK)])
        return carry

    lax.fori_loop(0, NCHUNK, chunk_body, 0)


_mesh = plsc.VectorSubcoreMesh(core_axis_name="c", subcore_axis_name="s")

_sc_call = pl.kernel(
    _body,
    out_type=jax.ShapeDtypeStruct((BATCH, HIDDEN), jnp.float32),
    mesh=_mesh,
    scratch_types=[
        pltpu.VMEM((N_FIELDS, CHUNK), jnp.int32),
        pltpu.VMEM((ROWS, HIDDEN), jnp.float32),
        pltpu.VMEM((CHUNK, HIDDEN), jnp.float32),
        pltpu.SemaphoreType.DMA,
    ],
    compiler_params=pltpu.CompilerParams(use_tc_tiling_on_sc=False),
)


def kernel(x, tables):
    x4 = x.astype(jnp.int32).reshape(N_FIELDS, NW, NCHUNK, CHUNK)
    return _sc_call(x4, tables)
